# baseline (speedup 1.0000x reference)
import functools
import math

import jax
import jax.numpy as jnp
from jax import lax
from jax.experimental import pallas as pl
from jax.experimental.pallas import tpu as pltpu
from jax.experimental.pallas import tpu_sc as plsc

D = 128
L = 16
NC = 2
NS = 16
NW = NC * NS
GROUP = 64
NBUF = 2
SCALE = math.sqrt(D)


def _make_sc_lookup(ng):
    mesh = plsc.VectorSubcoreMesh(
        core_axis_name="c", subcore_axis_name="s",
        num_cores=NC, num_subcores=NS)

    @functools.partial(
        pl.kernel,
        out_type=jax.ShapeDtypeStruct((NW, ng, GROUP, D), jnp.float32),
        mesh=mesh,
        scratch_types=[
            pltpu.VMEM((ng, GROUP), jnp.int32),
            pltpu.VMEM((NBUF, GROUP, D), jnp.float32),
            pltpu.VMEM((NBUF, GROUP, D), jnp.float32),
            pltpu.SemaphoreType.DMA,
            pltpu.SemaphoreType.DMA,
        ],
    )
    def body(idx_hbm, table_hbm, out_hbm, idx_v, rows_v, sout_v, gsem, osem):
        wid = lax.axis_index("s") * NC + lax.axis_index("c")
        pltpu.sync_copy(idx_hbm.at[wid], idx_v)

        for b in range(NBUF):
            pltpu.async_copy(table_hbm.at[idx_v.at[b]], rows_v.at[b], gsem)

        def scale_slot(b):
            def row(r, carry):
                for j in range(D // L):
                    sl = pl.ds(j * L, L)
                    sout_v[b, r, sl] = rows_v[b, r, sl] * SCALE
                return carry
            lax.fori_loop(0, GROUP, row, 0)

        def turn(t, b):
            g = t + b
            pltpu.make_async_copy(
                table_hbm.at[idx_v.at[b]], rows_v.at[b], gsem).wait()

            @pl.when(g >= NBUF)
            def _():
                pltpu.make_async_copy(
                    sout_v.at[b], out_hbm.at[wid, g], osem).wait()

            scale_slot(b)
            pltpu.async_copy(sout_v.at[b], out_hbm.at[wid, g], osem)

            @pl.when(g + NBUF < ng)
            def _():
                pltpu.async_copy(
                    table_hbm.at[idx_v.at[g + NBUF]], rows_v.at[b], gsem)

        def outer(i, carry):
            t = i * NBUF
            for b in range(NBUF):
                turn(t, b)
            return carry

        lax.fori_loop(0, ng // NBUF, outer, 0)

        for b in range(NBUF):
            pltpu.make_async_copy(
                sout_v.at[b], out_hbm.at[wid, 0], osem).wait()

    return body


def kernel(x, table):
    B, T = x.shape
    n = B * T
    assert n % (NW * GROUP) == 0
    ng = n // (NW * GROUP)
    idx = x.reshape(NW, ng, GROUP)
    if idx.dtype != jnp.int32:
        idx = idx.astype(jnp.int32)
    out = _make_sc_lookup(ng)(idx, table)
    return out.reshape(B, T, D)


# speedup vs baseline: 1.1762x; 1.1762x over previous
import functools
import math

import jax
import jax.numpy as jnp
from jax import lax
from jax.experimental import pallas as pl
from jax.experimental.pallas import tpu as pltpu
from jax.experimental.pallas import tpu_sc as plsc

D = 128
L = 16
NC = 2
NS = 16
NW = NC * NS
GROUP = 64
NBUF = 5
SCALE = math.sqrt(D)


def _make_sc_lookup(ng):
    mesh = plsc.VectorSubcoreMesh(
        core_axis_name="c", subcore_axis_name="s",
        num_cores=NC, num_subcores=NS)

    @functools.partial(
        pl.kernel,
        out_type=jax.ShapeDtypeStruct((NW, ng, GROUP, D), jnp.float32),
        mesh=mesh,
        scratch_types=[
            pltpu.VMEM((ng, GROUP), jnp.int32),
            pltpu.VMEM((NBUF, GROUP, D), jnp.float32),
            pltpu.VMEM((NBUF, GROUP, D), jnp.float32),
            pltpu.SemaphoreType.DMA,
            pltpu.SemaphoreType.DMA,
        ],
    )
    def body(idx_hbm, table_hbm, out_hbm, idx_v, rows_v, sout_v, gsem, osem):
        wid = lax.axis_index("s") * NC + lax.axis_index("c")
        pltpu.sync_copy(idx_hbm.at[wid], idx_v)

        for b in range(NBUF):
            pltpu.async_copy(table_hbm.at[idx_v.at[b]], rows_v.at[b], gsem)

        def scale_slot(b):
            def row(r, carry):
                for j in range(D // L):
                    sl = pl.ds(j * L, L)
                    sout_v[b, r, sl] = rows_v[b, r, sl] * SCALE
                return carry
            lax.fori_loop(0, GROUP, row, 0)

        def turn(t, b):
            g = t + b
            pltpu.make_async_copy(
                table_hbm.at[idx_v.at[b]], rows_v.at[b], gsem).wait()

            @pl.when(g >= NBUF)
            def _():
                pltpu.make_async_copy(
                    sout_v.at[b], out_hbm.at[wid, g], osem).wait()

            scale_slot(b)
            pltpu.async_copy(sout_v.at[b], out_hbm.at[wid, g], osem)

            @pl.when(g + NBUF < ng)
            def _():
                pltpu.async_copy(
                    table_hbm.at[idx_v.at[g + NBUF]], rows_v.at[b], gsem)

        def outer(i, carry):
            t = i * NBUF
            for b in range(NBUF):
                turn(t, b)
            return carry

        lax.fori_loop(0, ng // NBUF, outer, 0)

        for b in range(NBUF):
            pltpu.make_async_copy(
                sout_v.at[b], out_hbm.at[wid, 0], osem).wait()

    return body


def kernel(x, table):
    B, T = x.shape
    n = B * T
    assert n % (NW * GROUP) == 0
    ng = n // (NW * GROUP)
    idx = x.reshape(NW, ng, GROUP)
    if idx.dtype != jnp.int32:
        idx = idx.astype(jnp.int32)
    out = _make_sc_lookup(ng)(idx, table)
    return out.reshape(B, T, D)
